# baseline (speedup 1.0000x reference)
import dataclasses
import functools

import jax
import jax.numpy as jnp
from jax import lax
from jax.experimental import pallas as pl
from jax.experimental.pallas import tpu as pltpu
from jax.experimental.pallas import tpu_sc as plsc

NC = 2
NS = 16
L = 16
NW = NC * NS
B = 16384
D = 32
BPW = B // NW


def _sc_dot_gather(user_ids, item_ids, user_table, item_table):
    mesh = plsc.VectorSubcoreMesh(core_axis_name="c", subcore_axis_name="s")
    cp = pltpu.CompilerParams()
    if "needs_layout_passes" in pltpu.CompilerParams.__dataclass_fields__:
        cp = dataclasses.replace(cp, needs_layout_passes=False)
    if "use_tc_tiling_on_sc" in pltpu.CompilerParams.__dataclass_fields__:
        cp = dataclasses.replace(cp, use_tc_tiling_on_sc=False)

    @functools.partial(
        pl.kernel,
        compiler_params=cp,
        out_type=jax.ShapeDtypeStruct((B,), jnp.float32),
        mesh=mesh,
        scratch_types=[
            pltpu.VMEM((BPW,), jnp.int32),
            pltpu.VMEM((BPW,), jnp.int32),
            pltpu.VMEM((BPW, D), jnp.float32),
            pltpu.VMEM((BPW, D), jnp.float32),
            pltpu.VMEM((BPW,), jnp.float32),
            pltpu.SemaphoreType.DMA,
            pltpu.SemaphoreType.DMA,
        ],
    )
    def sc_kernel(uid_hbm, iid_hbm, utab_hbm, itab_hbm, out_hbm,
                  uidx_v, iidx_v, urows_v, irows_v, out_v, sem_u, sem_i):
        wid = lax.axis_index("s") * NC + lax.axis_index("c")
        base = wid * BPW
        pltpu.sync_copy(uid_hbm.at[pl.ds(base, BPW)], uidx_v)
        pltpu.sync_copy(iid_hbm.at[pl.ds(base, BPW)], iidx_v)
        cu = pltpu.async_copy(utab_hbm.at[uidx_v], urows_v, sem_u)
        ci = pltpu.async_copy(itab_hbm.at[iidx_v], irows_v, sem_i)
        cu.wait()
        ci.wait()

        lane = lax.iota(jnp.int32, L)
        last_lane = lane == (L - 1)

        @pl.loop(0, BPW)
        def _(i):
            u0 = urows_v[i, pl.ds(0, L)]
            u1 = urows_v[i, pl.ds(L, L)]
            v0 = irows_v[i, pl.ds(0, L)]
            v1 = irows_v[i, pl.ds(L, L)]
            c = plsc.cumsum(u0 * v0 + u1 * v1)
            idx = jnp.full((L,), i, jnp.int32)
            plsc.store_scatter(out_v, [idx], c, mask=last_lane)

        pltpu.sync_copy(out_v, out_hbm.at[pl.ds(base, BPW)])

    return sc_kernel(user_ids, item_ids, user_table, item_table)


def kernel(user_ids, item_ids, user_table, item_table):
    out = _sc_dot_gather(user_ids, item_ids, user_table, item_table)
    return out.reshape(B, 1)
